# Initial kernel scaffold; baseline (speedup 1.0000x reference)
import jax
import jax.numpy as jnp
from jax.experimental import pallas as pl


def kernel(x, pos, edge_index, edge_attr, batch_indices, params):
    raise NotImplementedError("write your pallas kernel here")


# speedup vs baseline: 2.7099x; 2.7099x over previous
import functools

import jax
import jax.numpy as jnp
from jax import lax
from jax.experimental import pallas as pl
from jax.experimental.pallas import tpu as pltpu
from jax.experimental.pallas import tpu_sc as plsc

N = 10000
E = 320000
D = 128
EDGE_DIM = 16
NUM_GRAPHS = 16

NC = 2
NS = 16
NW = NC * NS
EPW = E // NW
CH = 80
NCH = EPW // CH
RPT = N // NS
LANES = 16

_MESH = plsc.VectorSubcoreMesh(core_axis_name="c", subcore_axis_name="s")


def _wid():
    return lax.axis_index("s") * NC + lax.axis_index("c")


@functools.partial(
    pl.kernel,
    out_type=jax.ShapeDtypeStruct((E,), jnp.float32),
    mesh=_MESH,
    compiler_params=pltpu.CompilerParams(needs_layout_passes=False),
    scratch_types=[
        pltpu.VMEM((N,), jnp.float32),
        pltpu.VMEM((N,), jnp.float32),
        pltpu.VMEM((N,), jnp.float32),
        pltpu.VMEM((EPW,), jnp.int32),
        pltpu.VMEM((EPW,), jnp.int32),
        pltpu.VMEM((EPW,), jnp.float32),
    ],
)
def _sc_dist2(px_hbm, py_hbm, pz_hbm, src_hbm, dst_hbm, out_hbm,
              px, py, pz, sv, dv, ov):
    base = _wid() * EPW
    pltpu.sync_copy(px_hbm, px)
    pltpu.sync_copy(py_hbm, py)
    pltpu.sync_copy(pz_hbm, pz)
    pltpu.sync_copy(src_hbm.at[pl.ds(base, EPW)], sv)
    pltpu.sync_copy(dst_hbm.at[pl.ds(base, EPW)], dv)

    def body(i, carry):
        o = i * LANES
        s16 = sv[pl.ds(o, LANES)]
        d16 = dv[pl.ds(o, LANES)]
        rx = plsc.load_gather(px, [s16]) - plsc.load_gather(px, [d16])
        ry = plsc.load_gather(py, [s16]) - plsc.load_gather(py, [d16])
        rz = plsc.load_gather(pz, [s16]) - plsc.load_gather(pz, [d16])
        ov[pl.ds(o, LANES)] = rx * rx + ry * ry + rz * rz
        return carry

    lax.fori_loop(0, EPW // LANES, body, 0)
    pltpu.sync_copy(ov, out_hbm.at[pl.ds(base, EPW)])


@functools.partial(
    pl.kernel,
    out_type=jax.ShapeDtypeStruct((E, D), jnp.float32),
    mesh=_MESH,
    compiler_params=pltpu.CompilerParams(needs_layout_passes=False),
    scratch_types=[
        pltpu.VMEM((CH,), jnp.int32),
        pltpu.VMEM((CH, D), jnp.float32),
        pltpu.VMEM((CH, D), jnp.float32),
        pltpu.SemaphoreType.DMA,
    ],
)
def _sc_gather_add(a_hbm, b_hbm, src_hbm, dst_hbm, out_hbm, idxv, ra, rb, sem):
    base = _wid() * EPW

    def body(i, carry):
        off = base + i * CH
        pltpu.sync_copy(src_hbm.at[pl.ds(off, CH)], idxv)
        pltpu.async_copy(a_hbm.at[idxv], ra, sem).wait()
        pltpu.sync_copy(dst_hbm.at[pl.ds(off, CH)], idxv)
        pltpu.async_copy(b_hbm.at[idxv], rb, sem).wait()

        def add_row(r, c2):
            for c in range(D // LANES):
                sl = pl.ds(c * LANES, LANES)
                ra[r, sl] = ra[r, sl] + rb[r, sl]
            return c2

        lax.fori_loop(0, CH, add_row, 0)
        pltpu.sync_copy(ra, out_hbm.at[pl.ds(off, CH)])
        return carry

    lax.fori_loop(0, NCH, body, 0)


@functools.partial(
    pl.kernel,
    out_type=jax.ShapeDtypeStruct((NC, N, D), jnp.float32),
    mesh=_MESH,
    compiler_params=pltpu.CompilerParams(needs_layout_passes=False),
    scratch_types=[
        pltpu.VMEM((NCH, CH), jnp.int32),
        pltpu.VMEM((CH, D), jnp.float32),
        pltpu.VMEM_SHARED((N, D), jnp.float32),
    ],
)
def _sc_scatter(m2_hbm, dst3_hbm, zero_hbm, out_hbm, idx2, rows, shared):
    cid = lax.axis_index("c")
    sid = lax.axis_index("s")
    wid = sid * NC + cid
    row0 = sid * 624
    pltpu.sync_copy(zero_hbm.at[pl.ds(row0, 640)], shared.at[pl.ds(row0, 640)])
    pltpu.sync_copy(dst3_hbm.at[wid], idx2)
    plsc.subcore_barrier()

    def body(j, carry):
        pltpu.sync_copy(m2_hbm.at[pl.ds(wid * EPW + j * CH, CH)], rows)
        pltpu.sync_copy(rows, shared.at[idx2.at[j]], add=True)
        return carry

    lax.fori_loop(0, NCH, body, 0)
    plsc.subcore_barrier()
    pltpu.sync_copy(shared.at[pl.ds(row0, 640)],
                    out_hbm.at[cid].at[pl.ds(row0, 640)])


_BN = 400
_BE = 1000


def _tc_ab(h, w_src, w_dst, be1):
    def body(h_ref, ws_ref, wd_ref, b_ref, a_ref, b_out_ref):
        hb = h_ref[...]
        a_ref[...] = jnp.dot(hb, ws_ref[...], preferred_element_type=jnp.float32)
        b_out_ref[...] = (
            jnp.dot(hb, wd_ref[...], preferred_element_type=jnp.float32)
            + b_ref[...]
        )

    return pl.pallas_call(
        body,
        grid=(N // _BN,),
        in_specs=[
            pl.BlockSpec((_BN, D), lambda i: (i, 0)),
            pl.BlockSpec((D, D), lambda i: (0, 0)),
            pl.BlockSpec((D, D), lambda i: (0, 0)),
            pl.BlockSpec((1, D), lambda i: (0, 0)),
        ],
        out_specs=[
            pl.BlockSpec((_BN, D), lambda i: (i, 0)),
            pl.BlockSpec((_BN, D), lambda i: (i, 0)),
        ],
        out_shape=[
            jax.ShapeDtypeStruct((N, D), jnp.float32),
            jax.ShapeDtypeStruct((N, D), jnp.float32),
        ],
    )(h, w_src, w_dst, be1.reshape(1, D))


def _tc_edge(g, ea17, w17, we2, be2):
    def body(g_ref, ea_ref, w17_ref, w2_ref, b2_ref, o_ref):
        m1 = jax.nn.silu(
            g_ref[...]
            + jnp.dot(ea_ref[...], w17_ref[...], preferred_element_type=jnp.float32)
        )
        o_ref[...] = jax.nn.silu(
            jnp.dot(m1, w2_ref[...], preferred_element_type=jnp.float32)
            + b2_ref[...]
        )

    return pl.pallas_call(
        body,
        grid=(E // _BE,),
        in_specs=[
            pl.BlockSpec((_BE, D), lambda i: (i, 0)),
            pl.BlockSpec((_BE, EDGE_DIM + 1), lambda i: (i, 0)),
            pl.BlockSpec((EDGE_DIM + 1, D), lambda i: (0, 0)),
            pl.BlockSpec((D, D), lambda i: (0, 0)),
            pl.BlockSpec((1, D), lambda i: (0, 0)),
        ],
        out_specs=pl.BlockSpec((_BE, D), lambda i: (i, 0)),
        out_shape=jax.ShapeDtypeStruct((E, D), jnp.float32),
    )(g, ea17, w17, we2, be2.reshape(1, D))


def _tc_node(h, p, wh1_h, wh1_a, bh1, wh2, bh2):
    def body(h_ref, p_ref, wh_ref, wa_ref, b1_ref, w2_ref, b2_ref, o_ref):
        agg = p_ref[0] + p_ref[1]
        u = jax.nn.silu(
            jnp.dot(h_ref[...], wh_ref[...], preferred_element_type=jnp.float32)
            + jnp.dot(agg, wa_ref[...], preferred_element_type=jnp.float32)
            + b1_ref[...]
        )
        o_ref[...] = (
            jnp.dot(u, w2_ref[...], preferred_element_type=jnp.float32)
            + b2_ref[...]
        )

    return pl.pallas_call(
        body,
        grid=(N // _BN,),
        in_specs=[
            pl.BlockSpec((_BN, D), lambda i: (i, 0)),
            pl.BlockSpec((NC, _BN, D), lambda i: (0, i, 0)),
            pl.BlockSpec((D, D), lambda i: (0, 0)),
            pl.BlockSpec((D, D), lambda i: (0, 0)),
            pl.BlockSpec((1, D), lambda i: (0, 0)),
            pl.BlockSpec((D, D), lambda i: (0, 0)),
            pl.BlockSpec((1, D), lambda i: (0, 0)),
        ],
        out_specs=pl.BlockSpec((_BN, D), lambda i: (i, 0)),
        out_shape=jax.ShapeDtypeStruct((N, D), jnp.float32),
    )(h, p, wh1_h, wh1_a, bh1.reshape(1, D), wh2, bh2.reshape(1, D))


def _tc_head(h, batch3, wout, bout):
    grid = N // _BN

    def body(h_ref, b_ref, wo_ref, bo_ref, o_ref, sums, counts):
        i = pl.program_id(0)

        @pl.when(i == 0)
        def _():
            sums[...] = jnp.zeros_like(sums)
            counts[...] = jnp.zeros_like(counts)

        b = b_ref[...].reshape(1, _BN)
        gi = lax.broadcasted_iota(jnp.int32, (NUM_GRAPHS, _BN), 0)
        oh = (gi == b).astype(jnp.float32)
        sums[...] = sums[...] + jnp.dot(
            oh, h_ref[...], preferred_element_type=jnp.float32
        )
        counts[...] = counts[...] + jnp.sum(oh, axis=1, keepdims=True)

        @pl.when(i == grid - 1)
        def _():
            pooled = sums[...] / jnp.maximum(counts[...], 1.0)
            o_ref[...] = (
                jnp.dot(pooled, wo_ref[...], preferred_element_type=jnp.float32)
                + bo_ref[...]
            )

    return pl.pallas_call(
        body,
        grid=(grid,),
        in_specs=[
            pl.BlockSpec((_BN, D), lambda i: (i, 0)),
            pl.BlockSpec((1, 1, _BN), lambda i: (i, 0, 0)),
            pl.BlockSpec((D, 1), lambda i: (0, 0)),
            pl.BlockSpec((1, 1), lambda i: (0, 0)),
        ],
        out_specs=pl.BlockSpec((NUM_GRAPHS, 1), lambda i: (0, 0)),
        out_shape=jax.ShapeDtypeStruct((NUM_GRAPHS, 1), jnp.float32),
        scratch_shapes=[
            pltpu.VMEM((NUM_GRAPHS, D), jnp.float32),
            pltpu.VMEM((NUM_GRAPHS, 1), jnp.float32),
        ],
    )(h, batch3, wout, bout.reshape(1, 1))


def kernel(x, pos, edge_index, edge_attr, batch_indices, params):
    src = edge_index[0].astype(jnp.int32)
    dst = edge_index[1].astype(jnp.int32)
    px = pos[:, 0]
    py = pos[:, 1]
    pz = pos[:, 2]

    d2 = _sc_dist2(px, py, pz, src, dst)
    ea17 = jnp.concatenate([edge_attr, d2[:, None]], axis=1)
    dst3 = dst.reshape(NW, NCH, CH)
    zeros_nd = jnp.zeros((N, D), jnp.float32)

    h = x
    for lp in params["layers"]:
        we1 = lp["We1"]
        w_src = we1[0:D]
        w_dst = we1[D:2 * D]
        w17 = jnp.concatenate([we1[2 * D + 1:], we1[2 * D:2 * D + 1]], axis=0)
        a, b = _tc_ab(h, w_src, w_dst, lp["be1"])
        g = _sc_gather_add(a, b, src, dst)
        m2 = _tc_edge(g, ea17, w17, lp["We2"], lp["be2"])
        p = _sc_scatter(m2, dst3, zeros_nd)
        h = _tc_node(h, p, lp["Wh1"][:D], lp["Wh1"][D:], lp["bh1"],
                     lp["Wh2"], lp["bh2"])

    batch3 = batch_indices.astype(jnp.int32).reshape(N // _BN, 1, _BN)
    return _tc_head(h, batch3, params["Wout"], params["bout"])


# speedup vs baseline: 3.8890x; 1.4351x over previous
import functools

import jax
import jax.numpy as jnp
from jax import lax
from jax.experimental import pallas as pl
from jax.experimental.pallas import tpu as pltpu
from jax.experimental.pallas import tpu_sc as plsc

N = 10000
E = 320000
D = 128
EDGE_DIM = 16
NUM_GRAPHS = 16

NC = 2
NS = 16
NW = NC * NS
EPW = E // NW
CH = 40
NCH = EPW // CH
RPT = N // NS
LANES = 16

_MESH = plsc.VectorSubcoreMesh(core_axis_name="c", subcore_axis_name="s")


def _wid():
    return lax.axis_index("s") * NC + lax.axis_index("c")


@functools.partial(
    pl.kernel,
    out_type=jax.ShapeDtypeStruct((E,), jnp.float32),
    mesh=_MESH,
    compiler_params=pltpu.CompilerParams(needs_layout_passes=False),
    scratch_types=[
        pltpu.VMEM((N,), jnp.float32),
        pltpu.VMEM((N,), jnp.float32),
        pltpu.VMEM((N,), jnp.float32),
        pltpu.VMEM((EPW,), jnp.int32),
        pltpu.VMEM((EPW,), jnp.int32),
        pltpu.VMEM((EPW,), jnp.float32),
    ],
)
def _sc_dist2(px_hbm, py_hbm, pz_hbm, src_hbm, dst_hbm, out_hbm,
              px, py, pz, sv, dv, ov):
    base = _wid() * EPW
    pltpu.sync_copy(px_hbm, px)
    pltpu.sync_copy(py_hbm, py)
    pltpu.sync_copy(pz_hbm, pz)
    pltpu.sync_copy(src_hbm.at[pl.ds(base, EPW)], sv)
    pltpu.sync_copy(dst_hbm.at[pl.ds(base, EPW)], dv)

    def body(i, carry):
        o = i * LANES
        s16 = sv[pl.ds(o, LANES)]
        d16 = dv[pl.ds(o, LANES)]
        rx = plsc.load_gather(px, [s16]) - plsc.load_gather(px, [d16])
        ry = plsc.load_gather(py, [s16]) - plsc.load_gather(py, [d16])
        rz = plsc.load_gather(pz, [s16]) - plsc.load_gather(pz, [d16])
        ov[pl.ds(o, LANES)] = rx * rx + ry * ry + rz * rz
        return carry

    lax.fori_loop(0, EPW // LANES, body, 0)
    pltpu.sync_copy(ov, out_hbm.at[pl.ds(base, EPW)])


@functools.partial(
    pl.kernel,
    out_type=jax.ShapeDtypeStruct((E, D), jnp.float32),
    mesh=_MESH,
    compiler_params=pltpu.CompilerParams(needs_layout_passes=False),
    scratch_types=[
        pltpu.VMEM((EPW,), jnp.int32),
        pltpu.VMEM((EPW,), jnp.int32),
        pltpu.VMEM((CH, D), jnp.float32),
        pltpu.VMEM((CH, D), jnp.float32),
        pltpu.VMEM((CH, D), jnp.float32),
        pltpu.VMEM((CH, D), jnp.float32),
        pltpu.VMEM((CH, D), jnp.float32),
        pltpu.VMEM((CH, D), jnp.float32),
        pltpu.SemaphoreType.DMA,
        pltpu.SemaphoreType.DMA,
        pltpu.SemaphoreType.DMA,
        pltpu.SemaphoreType.DMA,
    ],
)
def _sc_gather_add(a_hbm, b_hbm, src_hbm, dst_hbm, out_hbm,
                   sv, dv, ra0, ra1, rb0, rb1, ro0, ro1,
                   sg0, sg1, sw0, sw1):
    base = _wid() * EPW
    pltpu.sync_copy(src_hbm.at[pl.ds(base, EPW)], sv)
    pltpu.sync_copy(dst_hbm.at[pl.ds(base, EPW)], dv)

    def start_gather(c, ra, rb, sg):
        pltpu.async_copy(a_hbm.at[sv.at[pl.ds(c * CH, CH)]], ra, sg)
        pltpu.async_copy(b_hbm.at[dv.at[pl.ds(c * CH, CH)]], rb, sg)

    def wait_gather(ra, rb, sg):
        pltpu.make_async_copy(a_hbm.at[pl.ds(0, CH)], ra, sg).wait()
        pltpu.make_async_copy(b_hbm.at[pl.ds(0, CH)], rb, sg).wait()

    def wait_write(ro, sw):
        pltpu.make_async_copy(ro, out_hbm.at[pl.ds(0, CH)], sw).wait()

    def add(ra, rb, ro):
        def row(r, c2):
            for c in range(D // LANES):
                sl = pl.ds(c * LANES, LANES)
                ro[r, sl] = ra[r, sl] + rb[r, sl]
            return c2

        lax.fori_loop(0, CH, row, 0)

    start_gather(0, ra0, rb0, sg0)
    start_gather(1, ra1, rb1, sg1)

    def slot(k, c, ra, rb, ro, sg, sw):
        wait_gather(ra, rb, sg)

        @pl.when(k > 0)
        def _():
            wait_write(ro, sw)

        add(ra, rb, ro)
        pltpu.async_copy(ro, out_hbm.at[pl.ds(base + c * CH, CH)], sw)

        @pl.when(c + 2 < NCH)
        def _():
            start_gather(c + 2, ra, rb, sg)

    def body(k, carry):
        slot(k, 2 * k, ra0, rb0, ro0, sg0, sw0)
        slot(k, 2 * k + 1, ra1, rb1, ro1, sg1, sw1)
        return carry

    lax.fori_loop(0, NCH // 2, body, 0)
    wait_write(ro0, sw0)
    wait_write(ro1, sw1)


@functools.partial(
    pl.kernel,
    out_type=jax.ShapeDtypeStruct((NC, N, D), jnp.float32),
    mesh=_MESH,
    compiler_params=pltpu.CompilerParams(needs_layout_passes=False),
    scratch_types=[
        pltpu.VMEM((NCH, CH), jnp.int32),
        pltpu.VMEM((CH, D), jnp.float32),
        pltpu.VMEM((CH, D), jnp.float32),
        pltpu.VMEM_SHARED((N, D), jnp.float32),
        pltpu.SemaphoreType.DMA,
        pltpu.SemaphoreType.DMA,
    ],
)
def _sc_scatter(m2_hbm, dst3_hbm, zero_hbm, out_hbm, idx2, r0, r1, shared,
                sl0, sl1):
    cid = lax.axis_index("c")
    sid = lax.axis_index("s")
    wid = sid * NC + cid
    base = wid * EPW
    row0 = sid * 624
    pltpu.sync_copy(zero_hbm.at[pl.ds(row0, 640)], shared.at[pl.ds(row0, 640)])
    pltpu.sync_copy(dst3_hbm.at[wid], idx2)
    plsc.subcore_barrier()

    def start_load(c, r, sl):
        pltpu.async_copy(m2_hbm.at[pl.ds(base + c * CH, CH)], r, sl)

    def wait_load(r, sl):
        pltpu.make_async_copy(m2_hbm.at[pl.ds(0, CH)], r, sl).wait()

    start_load(0, r0, sl0)
    start_load(1, r1, sl1)

    def slot(c, r, sl):
        wait_load(r, sl)
        pltpu.sync_copy(r, shared.at[idx2.at[c]], add=True)

        @pl.when(c + 2 < NCH)
        def _():
            start_load(c + 2, r, sl)

    def body(k, carry):
        slot(2 * k, r0, sl0)
        slot(2 * k + 1, r1, sl1)
        return carry

    lax.fori_loop(0, NCH // 2, body, 0)
    plsc.subcore_barrier()
    pltpu.sync_copy(shared.at[pl.ds(row0, 640)],
                    out_hbm.at[cid].at[pl.ds(row0, 640)])


_BN = 400
_BE = 1000


def _tc_ab(h, w_src, w_dst, be1):
    def body(h_ref, ws_ref, wd_ref, b_ref, a_ref, b_out_ref):
        hb = h_ref[...]
        a_ref[...] = jnp.dot(hb, ws_ref[...], preferred_element_type=jnp.float32)
        b_out_ref[...] = (
            jnp.dot(hb, wd_ref[...], preferred_element_type=jnp.float32)
            + b_ref[...]
        )

    return pl.pallas_call(
        body,
        grid=(N // _BN,),
        in_specs=[
            pl.BlockSpec((_BN, D), lambda i: (i, 0)),
            pl.BlockSpec((D, D), lambda i: (0, 0)),
            pl.BlockSpec((D, D), lambda i: (0, 0)),
            pl.BlockSpec((1, D), lambda i: (0, 0)),
        ],
        out_specs=[
            pl.BlockSpec((_BN, D), lambda i: (i, 0)),
            pl.BlockSpec((_BN, D), lambda i: (i, 0)),
        ],
        out_shape=[
            jax.ShapeDtypeStruct((N, D), jnp.float32),
            jax.ShapeDtypeStruct((N, D), jnp.float32),
        ],
    )(h, w_src, w_dst, be1.reshape(1, D))


def _tc_edge(g, ea17, w17, we2, be2):
    def body(g_ref, ea_ref, w17_ref, w2_ref, b2_ref, o_ref):
        m1 = jax.nn.silu(
            g_ref[...]
            + jnp.dot(ea_ref[...], w17_ref[...], preferred_element_type=jnp.float32)
        )
        o_ref[...] = jax.nn.silu(
            jnp.dot(m1, w2_ref[...], preferred_element_type=jnp.float32)
            + b2_ref[...]
        )

    return pl.pallas_call(
        body,
        grid=(E // _BE,),
        in_specs=[
            pl.BlockSpec((_BE, D), lambda i: (i, 0)),
            pl.BlockSpec((_BE, EDGE_DIM + 1), lambda i: (i, 0)),
            pl.BlockSpec((EDGE_DIM + 1, D), lambda i: (0, 0)),
            pl.BlockSpec((D, D), lambda i: (0, 0)),
            pl.BlockSpec((1, D), lambda i: (0, 0)),
        ],
        out_specs=pl.BlockSpec((_BE, D), lambda i: (i, 0)),
        out_shape=jax.ShapeDtypeStruct((E, D), jnp.float32),
    )(g, ea17, w17, we2, be2.reshape(1, D))


def _tc_node(h, p, wh1_h, wh1_a, bh1, wh2, bh2):
    def body(h_ref, p_ref, wh_ref, wa_ref, b1_ref, w2_ref, b2_ref, o_ref):
        agg = p_ref[0] + p_ref[1]
        u = jax.nn.silu(
            jnp.dot(h_ref[...], wh_ref[...], preferred_element_type=jnp.float32)
            + jnp.dot(agg, wa_ref[...], preferred_element_type=jnp.float32)
            + b1_ref[...]
        )
        o_ref[...] = (
            jnp.dot(u, w2_ref[...], preferred_element_type=jnp.float32)
            + b2_ref[...]
        )

    return pl.pallas_call(
        body,
        grid=(N // _BN,),
        in_specs=[
            pl.BlockSpec((_BN, D), lambda i: (i, 0)),
            pl.BlockSpec((NC, _BN, D), lambda i: (0, i, 0)),
            pl.BlockSpec((D, D), lambda i: (0, 0)),
            pl.BlockSpec((D, D), lambda i: (0, 0)),
            pl.BlockSpec((1, D), lambda i: (0, 0)),
            pl.BlockSpec((D, D), lambda i: (0, 0)),
            pl.BlockSpec((1, D), lambda i: (0, 0)),
        ],
        out_specs=pl.BlockSpec((_BN, D), lambda i: (i, 0)),
        out_shape=jax.ShapeDtypeStruct((N, D), jnp.float32),
    )(h, p, wh1_h, wh1_a, bh1.reshape(1, D), wh2, bh2.reshape(1, D))


def _tc_head(h, batch3, wout, bout):
    grid = N // _BN

    def body(h_ref, b_ref, wo_ref, bo_ref, o_ref, sums, counts):
        i = pl.program_id(0)

        @pl.when(i == 0)
        def _():
            sums[...] = jnp.zeros_like(sums)
            counts[...] = jnp.zeros_like(counts)

        b = b_ref[...].reshape(1, _BN)
        gi = lax.broadcasted_iota(jnp.int32, (NUM_GRAPHS, _BN), 0)
        oh = (gi == b).astype(jnp.float32)
        sums[...] = sums[...] + jnp.dot(
            oh, h_ref[...], preferred_element_type=jnp.float32
        )
        counts[...] = counts[...] + jnp.sum(oh, axis=1, keepdims=True)

        @pl.when(i == grid - 1)
        def _():
            pooled = sums[...] / jnp.maximum(counts[...], 1.0)
            o_ref[...] = (
                jnp.dot(pooled, wo_ref[...], preferred_element_type=jnp.float32)
                + bo_ref[...]
            )

    return pl.pallas_call(
        body,
        grid=(grid,),
        in_specs=[
            pl.BlockSpec((_BN, D), lambda i: (i, 0)),
            pl.BlockSpec((1, 1, _BN), lambda i: (i, 0, 0)),
            pl.BlockSpec((D, 1), lambda i: (0, 0)),
            pl.BlockSpec((1, 1), lambda i: (0, 0)),
        ],
        out_specs=pl.BlockSpec((NUM_GRAPHS, 1), lambda i: (0, 0)),
        out_shape=jax.ShapeDtypeStruct((NUM_GRAPHS, 1), jnp.float32),
        scratch_shapes=[
            pltpu.VMEM((NUM_GRAPHS, D), jnp.float32),
            pltpu.VMEM((NUM_GRAPHS, 1), jnp.float32),
        ],
    )(h, batch3, wout, bout.reshape(1, 1))


def kernel(x, pos, edge_index, edge_attr, batch_indices, params):
    src = edge_index[0].astype(jnp.int32)
    dst = edge_index[1].astype(jnp.int32)
    px = pos[:, 0]
    py = pos[:, 1]
    pz = pos[:, 2]

    d2 = _sc_dist2(px, py, pz, src, dst)
    ea17 = jnp.concatenate([edge_attr, d2[:, None]], axis=1)
    dst3 = dst.reshape(NW, NCH, CH)
    zeros_nd = jnp.zeros((N, D), jnp.float32)

    h = x
    for lp in params["layers"]:
        we1 = lp["We1"]
        w_src = we1[0:D]
        w_dst = we1[D:2 * D]
        w17 = jnp.concatenate([we1[2 * D + 1:], we1[2 * D:2 * D + 1]], axis=0)
        a, b = _tc_ab(h, w_src, w_dst, lp["be1"])
        g = _sc_gather_add(a, b, src, dst)
        m2 = _tc_edge(g, ea17, w17, lp["We2"], lp["be2"])
        p = _sc_scatter(m2, dst3, zeros_nd)
        h = _tc_node(h, p, lp["Wh1"][:D], lp["Wh1"][D:], lp["bh1"],
                     lp["Wh2"], lp["bh2"])

    batch3 = batch_indices.astype(jnp.int32).reshape(N // _BN, 1, _BN)
    return _tc_head(h, batch3, params["Wout"], params["bout"])
